# baseline (speedup 1.0000x reference)
import functools
import math

import jax
import jax.numpy as jnp
from jax import lax
from jax.experimental import pallas as pl
from jax.experimental.pallas import tpu as pltpu
from jax.experimental.pallas import tpu_sc as plsc


def kernel(token_ids, embedding, pe):
    B, T = token_ids.shape
    V, D = embedding.shape
    tok_flat = token_ids.reshape(B * T).astype(jnp.int32)
    pe_t = pe[0, :T, :]
    scale = math.sqrt(D)

    info = plsc.get_sparse_core_info()
    NC = info.num_cores
    NW = NC * info.num_subcores
    G = 40
    NTC = T // G
    BPW = B // NW

    mesh = plsc.VectorSubcoreMesh(core_axis_name="c", subcore_axis_name="s")

    @functools.partial(
        pl.kernel,
        mesh=mesh,
        out_type=jax.ShapeDtypeStruct((B * T, D), jnp.float32),
        scratch_types=[
            pltpu.VMEM((G,), jnp.int32),
            pltpu.VMEM((G, D), jnp.float32),
            pltpu.VMEM((G, D), jnp.float32),
            pltpu.SemaphoreType.DMA,
        ],
    )
    def emb_kernel(tok_hbm, emb_hbm, pe_hbm, out_hbm, idx_v, pe_v, rows_v, sem):
        wid = lax.axis_index("s") * NC + lax.axis_index("c")
        b0 = wid * BPW
        for tc in range(NTC):
            t0 = tc * G
            pltpu.sync_copy(pe_hbm.at[pl.ds(t0, G)], pe_v)

            def batch_body(i, _):
                row0 = (b0 + i) * T + t0
                pltpu.sync_copy(tok_hbm.at[pl.ds(row0, G)], idx_v)
                pltpu.async_copy(emb_hbm.at[idx_v], rows_v, sem).wait()

                def row_body(r, _):
                    for j in range(D // 16):
                        sl = pl.ds(j * 16, 16)
                        rows_v[r, sl] = rows_v[r, sl] * scale + pe_v[r, sl]
                    return 0

                lax.fori_loop(0, G, row_body, 0)
                pltpu.sync_copy(rows_v, out_hbm.at[pl.ds(row0, G)])
                return 0

            lax.fori_loop(0, BPW, batch_body, 0)

    out = emb_kernel(tok_flat, embedding, pe_t)
    return out.reshape(B, T, D)


# speedup vs baseline: 1.8793x; 1.8793x over previous
import functools
import math

import jax
import jax.numpy as jnp
from jax import lax
from jax.experimental import pallas as pl
from jax.experimental.pallas import tpu as pltpu
from jax.experimental.pallas import tpu_sc as plsc


def kernel(token_ids, embedding, pe):
    B, T = token_ids.shape
    V, D = embedding.shape
    tok_flat = token_ids.reshape(B * T).astype(jnp.int32)
    pe_t = pe[0, :T, :]
    scale = math.sqrt(D)

    info = plsc.get_sparse_core_info()
    NC = info.num_cores
    NW = NC * info.num_subcores
    G = 40
    NTC = T // G
    BPW = B // NW

    mesh = plsc.VectorSubcoreMesh(core_axis_name="c", subcore_axis_name="s")

    @functools.partial(
        pl.kernel,
        mesh=mesh,
        out_type=jax.ShapeDtypeStruct((B * T, D), jnp.float32),
        scratch_types=[
            pltpu.VMEM((G,), jnp.int32),
            pltpu.VMEM((G,), jnp.int32),
            pltpu.VMEM((G, D), jnp.float32),
            pltpu.VMEM((G, D), jnp.float32),
            pltpu.VMEM((G, D), jnp.float32),
            pltpu.VMEM((G, D), jnp.float32),
            pltpu.VMEM((G, D), jnp.float32),
            pltpu.SemaphoreType.DMA,
            pltpu.SemaphoreType.DMA,
            pltpu.SemaphoreType.DMA,
            pltpu.SemaphoreType.DMA,
        ],
    )
    def emb_kernel(tok_hbm, emb_hbm, pe_hbm, out_hbm,
                   idx0, idx1, in0, in1, out0, out1, pe_v,
                   sg0, sg1, ss0, ss1):
        wid = lax.axis_index("s") * NC + lax.axis_index("c")
        b0 = wid * BPW
        idx = (idx0, idx1)
        inb = (in0, in1)
        outb = (out0, out1)
        sg = (sg0, sg1)
        ss = (ss0, ss1)

        def start_gather(p, batch, t0):
            pltpu.sync_copy(tok_hbm.at[pl.ds(batch * T + t0, G)], idx[p])
            pltpu.async_copy(emb_hbm.at[idx[p]], inb[p], sg[p])

        def wait_gather(p):
            pltpu.make_async_copy(emb_hbm.at[idx[p]], inb[p], sg[p]).wait()

        def start_store(p, batch, t0):
            pltpu.async_copy(outb[p], out_hbm.at[pl.ds(batch * T + t0, G)],
                             ss[p])

        def wait_store(p):
            pltpu.make_async_copy(outb[p], out_hbm.at[pl.ds(0, G)],
                                  ss[p]).wait()

        def compute(p):
            def row_body(r, _):
                for j in range(D // 16):
                    sl = pl.ds(j * 16, 16)
                    outb[p][r, sl] = inb[p][r, sl] * scale + pe_v[r, sl]
                return 0
            lax.fori_loop(0, G, row_body, 0)

        for tc in range(NTC):
            t0 = tc * G
            pltpu.sync_copy(pe_hbm.at[pl.ds(t0, G)], pe_v)
            start_gather(0, b0 + 0, t0)
            start_gather(1, b0 + 1, t0)

            def slot_body(k, _):
                for p in range(2):
                    batch = b0 + 2 * k + p

                    @pl.when(k >= 1)
                    def _():
                        wait_store(p)

                    wait_gather(p)
                    compute(p)
                    start_store(p, batch, t0)

                    @pl.when(k < BPW // 2 - 1)
                    def _():
                        start_gather(p, batch + 2, t0)
                return 0

            lax.fori_loop(0, BPW // 2, slot_body, 0)
            wait_store(0)
            wait_store(1)

    out = emb_kernel(tok_flat, embedding, pe_t)
    return out.reshape(B, T, D)


# speedup vs baseline: 2.0678x; 1.1003x over previous
import functools
import math

import jax
import jax.numpy as jnp
from jax import lax
from jax.experimental import pallas as pl
from jax.experimental.pallas import tpu as pltpu
from jax.experimental.pallas import tpu_sc as plsc


def kernel(token_ids, embedding, pe):
    B, T = token_ids.shape
    V, D = embedding.shape
    tok_flat = token_ids.reshape(B * T).astype(jnp.int32)
    pe_t = pe[0, :T, :]
    scale = math.sqrt(D)

    info = plsc.get_sparse_core_info()
    NC = info.num_cores
    NW = NC * info.num_subcores
    G = 40
    NTC = T // G
    BPW = B // NW

    mesh = plsc.VectorSubcoreMesh(core_axis_name="c", subcore_axis_name="s")

    @functools.partial(
        pl.kernel,
        mesh=mesh,
        out_type=jax.ShapeDtypeStruct((B * T, D), jnp.float32),
        scratch_types=[
            pltpu.VMEM((BPW * T,), jnp.int32),
            pltpu.VMEM((G, D), jnp.float32),
            pltpu.VMEM((G, D), jnp.float32),
            pltpu.VMEM((G, D), jnp.float32),
            pltpu.VMEM((G, D), jnp.float32),
            pltpu.VMEM((G, D), jnp.float32),
            pltpu.SemaphoreType.DMA,
            pltpu.SemaphoreType.DMA,
            pltpu.SemaphoreType.DMA,
            pltpu.SemaphoreType.DMA,
        ],
    )
    def emb_kernel(tok_hbm, emb_hbm, pe_hbm, out_hbm,
                   idx_all, in0, in1, out0, out1, pe_v,
                   sg0, sg1, ss0, ss1):
        wid = lax.axis_index("s") * NC + lax.axis_index("c")
        b0 = wid * BPW
        inb = (in0, in1)
        outb = (out0, out1)
        sg = (sg0, sg1)
        ss = (ss0, ss1)
        pltpu.sync_copy(tok_hbm.at[pl.ds(b0 * T, BPW * T)], idx_all)

        def start_gather(p, batch, t0):
            off = (batch - b0) * T + t0
            pltpu.async_copy(emb_hbm.at[idx_all.at[pl.ds(off, G)]],
                             inb[p], sg[p])

        def wait_gather(p):
            pltpu.make_async_copy(emb_hbm.at[idx_all.at[pl.ds(0, G)]],
                                  inb[p], sg[p]).wait()

        def start_store(p, batch, t0):
            pltpu.async_copy(outb[p], out_hbm.at[pl.ds(batch * T + t0, G)],
                             ss[p])

        def wait_store(p):
            pltpu.make_async_copy(outb[p], out_hbm.at[pl.ds(0, G)],
                                  ss[p]).wait()

        def compute(p):
            def row_body(r, _):
                for j in range(D // 16):
                    sl = pl.ds(j * 16, 16)
                    outb[p][r, sl] = inb[p][r, sl] * scale + pe_v[r, sl]
                return 0
            lax.fori_loop(0, G, row_body, 0)

        for tc in range(NTC):
            t0 = tc * G
            pltpu.sync_copy(pe_hbm.at[pl.ds(t0, G)], pe_v)
            start_gather(0, b0 + 0, t0)
            start_gather(1, b0 + 1, t0)

            def slot_body(k, _):
                for p in range(2):
                    batch = b0 + 2 * k + p

                    @pl.when(k >= 1)
                    def _():
                        wait_store(p)

                    wait_gather(p)
                    compute(p)
                    start_store(p, batch, t0)

                    @pl.when(k < BPW // 2 - 1)
                    def _():
                        start_gather(p, batch + 2, t0)
                return 0

            lax.fori_loop(0, BPW // 2, slot_body, 0)
            wait_store(0)
            wait_store(1)

    out = emb_kernel(tok_flat, embedding, pe_t)
    return out.reshape(B, T, D)


# speedup vs baseline: 2.1994x; 1.0636x over previous
import functools
import math

import jax
import jax.numpy as jnp
from jax import lax
from jax.experimental import pallas as pl
from jax.experimental.pallas import tpu as pltpu
from jax.experimental.pallas import tpu_sc as plsc


def kernel(token_ids, embedding, pe):
    B, T = token_ids.shape
    V, D = embedding.shape
    tok_flat = token_ids.reshape(B * T).astype(jnp.int32)
    pe_t = pe[0, :T, :]
    scale = math.sqrt(D)

    info = plsc.get_sparse_core_info()
    NC = info.num_cores
    NW = NC * info.num_subcores
    G = 40
    NTC = T // G
    BPW = B // NW
    NSLOT = NTC * BPW
    NB = 5

    mesh = plsc.VectorSubcoreMesh(core_axis_name="c", subcore_axis_name="s")

    @functools.partial(
        pl.kernel,
        mesh=mesh,
        out_type=jax.ShapeDtypeStruct((B * T, D), jnp.float32),
        scratch_types=[
            pltpu.VMEM((BPW * T,), jnp.int32),
            pltpu.VMEM((G, D), jnp.float32),
        ] + [pltpu.VMEM((G, D), jnp.float32) for _ in range(NB)]
          + [pltpu.SemaphoreType.DMA for _ in range(2 * NB)],
    )
    def emb_kernel(tok_hbm, emb_hbm, pe_hbm, out_hbm, idx_all, pe_v, *rest):
        buf = rest[:NB]
        sg = rest[NB:2 * NB]
        ss = rest[2 * NB:3 * NB]
        wid = lax.axis_index("s") * NC + lax.axis_index("c")
        b0 = wid * BPW
        pltpu.sync_copy(tok_hbm.at[pl.ds(b0 * T, BPW * T)], idx_all)

        def slot_off(s):
            chunk = s // BPW
            blocal = s % BPW
            t0 = chunk * G
            return blocal * T + t0, (b0 + blocal) * T + t0

        def start_gather(p, s):
            off, _ = slot_off(s)
            pltpu.async_copy(emb_hbm.at[idx_all.at[pl.ds(off, G)]],
                             buf[p], sg[p])

        def wait_gather(p):
            pltpu.make_async_copy(emb_hbm.at[idx_all.at[pl.ds(0, G)]],
                                  buf[p], sg[p]).wait()

        def start_store(p, s):
            _, row0 = slot_off(s)
            pltpu.async_copy(buf[p], out_hbm.at[pl.ds(row0, G)], ss[p])

        def wait_store(p):
            pltpu.make_async_copy(buf[p], out_hbm.at[pl.ds(0, G)],
                                  ss[p]).wait()

        def compute(p):
            def row_body(r, _):
                for j in range(D // 16):
                    sl = pl.ds(j * 16, 16)
                    buf[p][r, sl] = buf[p][r, sl] * scale + pe_v[r, sl]
                return 0
            lax.fori_loop(0, G, row_body, 0)

        pltpu.sync_copy(pe_hbm.at[pl.ds(0, G)], pe_v)
        for p in range(3):
            start_gather(p, p)

        def body(k, _):
            for u in range(NB):
                s = k * NB + u
                p = u

                @pl.when(jnp.logical_and(s % BPW == 0, s > 0))
                def _():
                    pltpu.sync_copy(
                        pe_hbm.at[pl.ds((s // BPW) * G, G)], pe_v)

                wait_gather(p)
                compute(p)
                start_store(p, s)

                q = (u + 3) % NB

                @pl.when(s >= 2)
                def _():
                    wait_store(q)

                @pl.when(s + 3 < NSLOT)
                def _():
                    start_gather(q, s + 3)
            return 0

        lax.fori_loop(0, NSLOT // NB, body, 0)
        wait_store((NSLOT - 2) % NB)
        wait_store((NSLOT - 1) % NB)

    out = emb_kernel(tok_flat, embedding, pe_t)
    return out.reshape(B, T, D)
